# Initial kernel scaffold; baseline (speedup 1.0000x reference)
import jax
import jax.numpy as jnp
from jax import lax
from jax.experimental import pallas as pl
from jax.experimental.pallas import tpu as pltpu
from jax.experimental.pallas import tpu_sc as plsc

N = 10000
E = 320000
IN_DIM = 128
HEADS = 4
OUT_DIM = 32
FDIM = HEADS * OUT_DIM

NC = 2
NS = 16
WORKERS = NC * NS

E_REAL = E + N
CHUNK = 128
CPT = 81
TILE_E = CPT * CHUNK
E_PAD = WORKERS * TILE_E
ROWS_PER_TILE = N // NS
ACC_W = FDIM + 16
NEG_SLOPE = 0.2


def _prep_body(x_ref, wg_ref, ws_ref, as_ref, ad_ref, b_ref,
               h_ref, tab_ref, base_ref):
    x = x_ref[...]
    h = jnp.dot(x, wg_ref[...], preferred_element_type=jnp.float32)
    h_ref[...] = h
    base_ref[...] = (
        jnp.dot(x, ws_ref[...], preferred_element_type=jnp.float32)
        + b_ref[...][None, :]
    )
    cols = []
    for att in (as_ref, ad_ref):
        for hd in range(HEADS):
            seg = h[:, hd * OUT_DIM:(hd + 1) * OUT_DIM]
            v = att[0, hd, :][None, :]
            cols.append(jnp.sum(seg * v, axis=1, keepdims=True))
    tab_ref[...] = jnp.concatenate(cols, axis=1)


def _edge_body(src_hbm, dst_hbm, h_hbm, tab_hbm, acc_hbm,
               tab_v, src_v, dst_v, rows_v, msg_v, wbuf_v, acc_sh, sem):
    c = lax.axis_index("c")
    s = lax.axis_index("s")
    zero16 = jnp.zeros((16,), jnp.float32)
    iota16 = lax.iota(jnp.int32, 16)

    def _zrow(i, _):
        for j in range(ACC_W // 16):
            msg_v[i, pl.ds(j * 16, 16)] = zero16
        return 0
    lax.fori_loop(0, CHUNK, _zrow, 0)
    for t in range(5):
        pltpu.sync_copy(msg_v.at[pl.ds(0, 125)],
                        acc_sh.at[pl.ds(s * ROWS_PER_TILE + t * 125, 125)])

    pltpu.sync_copy(tab_hbm, tab_v)
    plsc.subcore_barrier()

    wid = c * NS + s
    tile_base = wid * TILE_E

    def _chunk(g, _):
        base = tile_base + g * CHUNK
        pltpu.sync_copy(src_hbm.at[pl.ds(base, CHUNK)], src_v)
        pltpu.sync_copy(dst_hbm.at[pl.ds(base, CHUNK)], dst_v)
        pltpu.async_copy(h_hbm.at[src_v], rows_v, sem).wait()

        for grp in range(CHUNK // 16):
            sidx = src_v[pl.ds(grp * 16, 16)]
            didx = dst_v[pl.ds(grp * 16, 16)]
            gid = base + grp * 16 + iota16
            valid = gid < E_REAL
            for hd in range(HEADS):
                a_s = plsc.load_gather(
                    tab_v, [sidx, jnp.full((16,), hd, jnp.int32)])
                a_d = plsc.load_gather(
                    tab_v, [didx, jnp.full((16,), HEADS + hd, jnp.int32)])
                e = a_s + a_d
                e = jnp.maximum(e, NEG_SLOPE * e)
                w = jnp.where(valid, jnp.exp(e), 0.0)
                wbuf_v[hd, pl.ds(grp * 16, 16)] = w

        def _mul(k, _):
            for hd in range(HEADS):
                wk = wbuf_v[hd, k]
                wv = jnp.full((16,), wk, jnp.float32)
                for j in range(2):
                    col = hd * OUT_DIM + j * 16
                    msg_v[k, pl.ds(col, 16)] = rows_v[k, pl.ds(col, 16)] * wv
            wrow = plsc.load_gather(
                wbuf_v, [jnp.minimum(iota16, HEADS - 1),
                         jnp.full((16,), k, jnp.int32)])
            msg_v[k, pl.ds(FDIM, 16)] = jnp.where(iota16 < HEADS, wrow, 0.0)
            return 0
        lax.fori_loop(0, CHUNK, _mul, 0)

        pltpu.sync_copy(msg_v, acc_sh.at[dst_v], add=True)
        return 0

    lax.fori_loop(0, CPT, _chunk, 0)
    plsc.subcore_barrier()

    row0 = s * ROWS_PER_TILE
    pltpu.sync_copy(acc_sh.at[pl.ds(row0, ROWS_PER_TILE)],
                    acc_hbm.at[c, pl.ds(row0, ROWS_PER_TILE)])


def _final_body(acc_ref, base_ref, out_ref):
    a = acc_ref[0] + acc_ref[1]
    den = a[:, FDIM:FDIM + HEADS] + 1e-16
    cols = []
    for hd in range(HEADS):
        feat = a[:, hd * OUT_DIM:(hd + 1) * OUT_DIM]
        cols.append(feat / den[:, hd:hd + 1])
    o = jnp.concatenate(cols, axis=1) + base_ref[...]
    out_ref[...] = jnp.where(o > 0, o, jnp.expm1(o))


def kernel(x, edge_index, W_gat, att_src, att_dst, bias, W_skip):
    blk = 1000
    grid = N // blk
    h, tab, base = pl.pallas_call(
        _prep_body,
        grid=(grid,),
        in_specs=[
            pl.BlockSpec((blk, IN_DIM), lambda i: (i, 0)),
            pl.BlockSpec((IN_DIM, FDIM), lambda i: (0, 0)),
            pl.BlockSpec((IN_DIM, FDIM), lambda i: (0, 0)),
            pl.BlockSpec((1, HEADS, OUT_DIM), lambda i: (0, 0, 0)),
            pl.BlockSpec((1, HEADS, OUT_DIM), lambda i: (0, 0, 0)),
            pl.BlockSpec((FDIM,), lambda i: (0,)),
        ],
        out_specs=[
            pl.BlockSpec((blk, FDIM), lambda i: (i, 0)),
            pl.BlockSpec((blk, 2 * HEADS), lambda i: (i, 0)),
            pl.BlockSpec((blk, FDIM), lambda i: (i, 0)),
        ],
        out_shape=[
            jax.ShapeDtypeStruct((N, FDIM), jnp.float32),
            jax.ShapeDtypeStruct((N, 2 * HEADS), jnp.float32),
            jax.ShapeDtypeStruct((N, FDIM), jnp.float32),
        ],
    )(x, W_gat, W_skip, att_src, att_dst, bias)

    loop = jnp.arange(N, dtype=jnp.int32)
    pad = jnp.zeros((E_PAD - E_REAL,), jnp.int32)
    src = jnp.concatenate([edge_index[0], loop, pad])
    dst = jnp.concatenate([edge_index[1], loop, pad])

    mesh = plsc.VectorSubcoreMesh(
        core_axis_name="c", subcore_axis_name="s",
        num_cores=NC, num_subcores=NS)
    edge_fn = pl.kernel(
        _edge_body,
        out_type=jax.ShapeDtypeStruct((NC, N, ACC_W), jnp.float32),
        mesh=mesh,
        scratch_types=[
            pltpu.VMEM((N, 2 * HEADS), jnp.float32),
            pltpu.VMEM((CHUNK,), jnp.int32),
            pltpu.VMEM((CHUNK,), jnp.int32),
            pltpu.VMEM((CHUNK, FDIM), jnp.float32),
            pltpu.VMEM((CHUNK, ACC_W), jnp.float32),
            pltpu.VMEM((HEADS, CHUNK), jnp.float32),
            pltpu.VMEM_SHARED((N, ACC_W), jnp.float32),
            pltpu.SemaphoreType.DMA,
        ],
    )
    acc = edge_fn(src, dst, h, tab)

    out = pl.pallas_call(
        _final_body,
        grid=(grid,),
        in_specs=[
            pl.BlockSpec((NC, blk, ACC_W), lambda i: (0, i, 0)),
            pl.BlockSpec((blk, FDIM), lambda i: (i, 0)),
        ],
        out_specs=pl.BlockSpec((blk, FDIM), lambda i: (i, 0)),
        out_shape=jax.ShapeDtypeStruct((N, FDIM), jnp.float32),
    )(acc, base)
    return out


# speedup vs baseline: 44.3391x; 44.3391x over previous
import jax
import jax.numpy as jnp
from jax import lax
from jax.experimental import pallas as pl
from jax.experimental.pallas import tpu as pltpu
from jax.experimental.pallas import tpu_sc as plsc

N = 10000
E = 320000
IN_DIM = 128
HEADS = 4
OUT_DIM = 32
FDIM = HEADS * OUT_DIM
HDIM = FDIM // 2

NC = 2
NS = 16

E_REAL = E + N
CHUNK = 64
CPT = 324
TILE_E = CPT * CHUNK
E_PAD = NS * TILE_E
NPAD = 10240
NACC = 10000
RPTA = 624
DPACK = 4
DROWS = NPAD // DPACK
DACC = 2504
DTL = 152
NEG_SLOPE = 0.2


def _prep_body(x_ref, wg_ref, ws_ref, as_ref, ad_ref, b_ref,
               hs_ref, tab_ref, base_ref):
    x = x_ref[...]
    h = jnp.dot(x, wg_ref[...], preferred_element_type=jnp.float32)
    skip = (jnp.dot(x, ws_ref[...], preferred_element_type=jnp.float32)
            + b_ref[...][None, :])
    acol = []
    for att in (as_ref, ad_ref):
        for hd in range(HEADS):
            seg = h[:, hd * OUT_DIM:(hd + 1) * OUT_DIM]
            v = att[0, hd, :][None, :]
            acol.append(jnp.sum(seg * v, axis=1, keepdims=True))
    base_ref[...] = skip
    for c in range(NC):
        hs_ref[c] = h[:, c * HDIM:(c + 1) * HDIM]
        tab_ref[c] = jnp.concatenate(
            [acol[2 * c], acol[2 * c + 1],
             acol[HEADS + 2 * c], acol[HEADS + 2 * c + 1]], axis=1)


def _edge_body(pk_hbm, hs_hbm, tab_hbm, feat_hbm, den_hbm,
               tab_v, pk_v, dst_v, dst4_v, src2_v, rows_v, dmsg_v,
               wbuf_v, feat_sh, den_sh, sem):
    c = lax.axis_index("c")
    s = lax.axis_index("s")
    zero16 = jnp.zeros((16,), jnp.float32)
    iota16 = lax.iota(jnp.int32, 16)

    def _zrow(i, _):
        for j in range(HDIM // 16):
            rows_v[i, pl.ds(j * 16, 16)] = zero16
        return 0
    lax.fori_loop(0, CHUNK, _zrow, 0)
    for i in range(CHUNK * 8 // 16):
        p = i * 16 + iota16
        plsc.store_scatter(
            dmsg_v, [lax.shift_right_logical(p, 3), p & 7], zero16)
    def _zblk(t, _):
        pltpu.sync_copy(rows_v, feat_sh.at[pl.ds(s * RPTA + t * CHUNK, CHUNK)])
        return 0
    lax.fori_loop(0, 9, _zblk, 0)
    pltpu.sync_copy(rows_v.at[pl.ds(0, RPTA - 9 * CHUNK)],
                    feat_sh.at[pl.ds(s * RPTA + 9 * CHUNK, RPTA - 9 * CHUNK)])
    pltpu.sync_copy(dmsg_v, den_sh.at[pl.ds(s * DTL, CHUNK)])
    pltpu.sync_copy(dmsg_v, den_sh.at[pl.ds(s * DTL + CHUNK, CHUNK)])
    pltpu.sync_copy(dmsg_v.at[pl.ds(0, DTL - 2 * CHUNK)],
                    den_sh.at[pl.ds(s * DTL + 2 * CHUNK, DTL - 2 * CHUNK)])

    @pl.when(s == NS - 1)
    def _zero_tail():
        pltpu.sync_copy(rows_v.at[pl.ds(0, NACC - NS * RPTA)],
                        feat_sh.at[pl.ds(NS * RPTA, NACC - NS * RPTA)])
        pltpu.sync_copy(dmsg_v.at[pl.ds(0, DACC - NS * DTL)],
                        den_sh.at[pl.ds(NS * DTL, DACC - NS * DTL)])

    pltpu.sync_copy(tab_hbm.at[c], tab_v)
    plsc.subcore_barrier()

    tile_base = s * TILE_E
    row_off = c * N

    def _chunk(g, _):
        base = tile_base + g * CHUNK
        pltpu.sync_copy(pk_hbm.at[pl.ds(base, CHUNK)], pk_v)

        for grp in range(CHUNK // 16):
            p = pk_v[pl.ds(grp * 16, 16)]
            sidx = p & 0xFFFF
            didx = lax.shift_right_logical(p, 16)
            src2_v[pl.ds(grp * 16, 16)] = sidx + row_off
            dst_v[pl.ds(grp * 16, 16)] = didx
            dst4_v[pl.ds(grp * 16, 16)] = lax.shift_right_logical(didx, 2)
            gid = base + grp * 16 + iota16
            valid = gid < E_REAL
            row_idx = grp * 16 + iota16
            lane0 = (didx & 3) * 2
            for hd in range(2):
                a_s = plsc.load_gather(
                    tab_v, [sidx, jnp.full((16,), hd, jnp.int32)])
                a_d = plsc.load_gather(
                    tab_v, [didx, jnp.full((16,), 2 + hd, jnp.int32)])
                e = a_s + a_d
                e = jnp.maximum(e, NEG_SLOPE * e)
                w = jnp.where(valid, jnp.exp(e), 0.0)
                wbuf_v[hd, pl.ds(grp * 16, 16)] = w
                plsc.store_scatter(dmsg_v, [row_idx, lane0 + hd], w)

        gath = pltpu.async_copy(hs_hbm.at[src2_v], rows_v, sem)
        pltpu.sync_copy(dmsg_v, den_sh.at[dst4_v], add=True)
        for grp in range(CHUNK // 16):
            didx = dst_v[pl.ds(grp * 16, 16)]
            row_idx = grp * 16 + iota16
            lane0 = (didx & 3) * 2
            for hd in range(2):
                plsc.store_scatter(dmsg_v, [row_idx, lane0 + hd], zero16)
        gath.wait()

        def _mul(k, _):
            for hd in range(2):
                wk = wbuf_v[hd, pl.ds(k, 16)][0]
                wv = jnp.full((16,), wk, jnp.float32)
                for j in range(2):
                    col = hd * OUT_DIM + j * 16
                    rows_v[k, pl.ds(col, 16)] = rows_v[k, pl.ds(col, 16)] * wv
            return 0
        lax.fori_loop(0, CHUNK, _mul, 0)

        pltpu.sync_copy(rows_v, feat_sh.at[dst_v], add=True)
        return 0

    lax.fori_loop(0, CPT, _chunk, 0)
    plsc.subcore_barrier()

    pltpu.sync_copy(feat_sh.at[pl.ds(s * RPTA, RPTA)],
                    feat_hbm.at[c, pl.ds(s * RPTA, RPTA)])
    pltpu.sync_copy(den_sh.at[pl.ds(s * DTL, DTL)],
                    den_hbm.at[c, pl.ds(s * DTL, DTL)])

    @pl.when(s == NS - 1)
    def _dump_tail():
        pltpu.sync_copy(feat_sh.at[pl.ds(NS * RPTA, NACC - NS * RPTA)],
                        feat_hbm.at[c, pl.ds(NS * RPTA, NACC - NS * RPTA)])
        pltpu.sync_copy(den_sh.at[pl.ds(NS * DTL, DACC - NS * DTL)],
                        den_hbm.at[c, pl.ds(NS * DTL, DACC - NS * DTL)])


def kernel(x, edge_index, W_gat, att_src, att_dst, bias, W_skip):
    blk = 1000
    grid = N // blk
    hs, tab, base = pl.pallas_call(
        _prep_body,
        grid=(grid,),
        in_specs=[
            pl.BlockSpec((blk, IN_DIM), lambda i: (i, 0)),
            pl.BlockSpec((IN_DIM, FDIM), lambda i: (0, 0)),
            pl.BlockSpec((IN_DIM, FDIM), lambda i: (0, 0)),
            pl.BlockSpec((1, HEADS, OUT_DIM), lambda i: (0, 0, 0)),
            pl.BlockSpec((1, HEADS, OUT_DIM), lambda i: (0, 0, 0)),
            pl.BlockSpec((FDIM,), lambda i: (0,)),
        ],
        out_specs=[
            pl.BlockSpec((NC, blk, HDIM), lambda i: (0, i, 0)),
            pl.BlockSpec((NC, blk, HEADS), lambda i: (0, i, 0)),
            pl.BlockSpec((blk, FDIM), lambda i: (i, 0)),
        ],
        out_shape=[
            jax.ShapeDtypeStruct((NC, N, HDIM), jnp.float32),
            jax.ShapeDtypeStruct((NC, N, HEADS), jnp.float32),
            jax.ShapeDtypeStruct((NPAD, FDIM), jnp.float32),
        ],
    )(x, W_gat, W_skip, att_src, att_dst, bias)
    hs2 = hs.reshape(NC * N, HDIM)

    loop = jnp.arange(N, dtype=jnp.int32)
    pad = jnp.zeros((E_PAD - E_REAL,), jnp.int32)
    src = jnp.concatenate([edge_index[0], loop, pad])
    dst = jnp.concatenate([edge_index[1], loop, pad])
    pk = (dst << 16) | src

    mesh = plsc.VectorSubcoreMesh(
        core_axis_name="c", subcore_axis_name="s",
        num_cores=NC, num_subcores=NS)
    edge_fn = pl.kernel(
        _edge_body,
        out_type=[
            jax.ShapeDtypeStruct((NC, NPAD, HDIM), jnp.float32),
            jax.ShapeDtypeStruct((NC, DROWS, 8), jnp.float32),
        ],
        mesh=mesh,
        compiler_params=pltpu.CompilerParams(
            needs_layout_passes=False, use_tc_tiling_on_sc=False),
        scratch_types=[
            pltpu.VMEM((N, 4), jnp.float32),
            pltpu.VMEM((CHUNK,), jnp.int32),
            pltpu.VMEM((CHUNK,), jnp.int32),
            pltpu.VMEM((CHUNK,), jnp.int32),
            pltpu.VMEM((CHUNK,), jnp.int32),
            pltpu.VMEM((CHUNK, HDIM), jnp.float32),
            pltpu.VMEM((CHUNK, 8), jnp.float32),
            pltpu.VMEM((2, CHUNK + 16), jnp.float32),
            pltpu.VMEM_SHARED((NACC, HDIM), jnp.float32),
            pltpu.VMEM_SHARED((DACC, 8), jnp.float32),
            pltpu.SemaphoreType.DMA,
        ],
    )
    feat, den = edge_fn(pk, hs2, tab)

    fblk = 1024
    out = pl.pallas_call(
        _final_body,
        grid=(NPAD // fblk,),
        in_specs=[
            pl.BlockSpec((NC, fblk, HDIM), lambda i: (0, i, 0)),
            pl.BlockSpec((NC, fblk // DPACK, 8), lambda i: (0, i, 0)),
            pl.BlockSpec((fblk, FDIM), lambda i: (i, 0)),
        ],
        out_specs=pl.BlockSpec((fblk, FDIM), lambda i: (i, 0)),
        out_shape=jax.ShapeDtypeStruct((NPAD, FDIM), jnp.float32),
    )(feat, den, base)
    return out[:N]


def _final_body(feat_ref, den_ref, base_ref, out_ref):
    blk = feat_ref.shape[1]
    rows = blk // DPACK
    rsel = (lax.broadcasted_iota(jnp.int32, (blk, rows), 0) // DPACK
            == lax.broadcasted_iota(jnp.int32, (blk, rows), 1))
    rsel = rsel.astype(jnp.float32)
    lane_of_n = (lax.broadcasted_iota(jnp.int32, (blk, 8), 0) % DPACK) * 2
    lane_id = lax.broadcasted_iota(jnp.int32, (blk, 8), 1)
    cols = []
    for c in range(NC):
        expand = jnp.dot(rsel, den_ref[c],
                         preferred_element_type=jnp.float32)
        for hd in range(2):
            den = jnp.sum(
                jnp.where(lane_id == lane_of_n + hd, expand, 0.0),
                axis=1, keepdims=True) + 1e-16
            seg = feat_ref[c][:, hd * OUT_DIM:(hd + 1) * OUT_DIM]
            cols.append(seg / den)
    o = jnp.concatenate(cols, axis=1) + base_ref[...]
    out_ref[...] = jnp.where(o > 0, o, jnp.exp(jnp.minimum(o, 0.0)) - 1.0)
